# baseline (speedup 1.0000x reference)
import jax
import jax.numpy as jnp
from jax import lax
from jax.experimental import pallas as pl
from jax.experimental.pallas import tpu as pltpu
from jax.experimental.pallas import tpu_sc as plsc

N = 10000
H = 128
HH = H // 2
TGT = 4
LAYERS = 3

NTILE = 16
NCORE = 2
NWORK = NTILE * NCORE

RPT = 632
RPT_LAST = N - (NTILE - 1) * RPT

CH = 128
CHD = 128
NPAD = N + 8

RELSPEC = [
    ("topo_out", "block", 160000),
    ("topo_in", "net", 160000),
    ("grid_edge", "grid", 64000),
    ("geom_loc", "block", 40000),
    ("geom_bb", "net", 160000),
]

_MESH = plsc.VectorSubcoreMesh(core_axis_name="c", subcore_axis_name="s")
_HIGH = jax.lax.Precision.HIGHEST


def _zero_slab(zsrc, acc_sh, s):
    row0 = s * RPT

    @pl.when(s < NTILE - 1)
    def _():
        pltpu.sync_copy(zsrc.at[pl.ds(0, RPT)], acc_sh.at[pl.ds(row0, RPT)])

    @pl.when(s == NTILE - 1)
    def _():
        pltpu.sync_copy(zsrc.at[pl.ds(0, RPT_LAST)],
                        acc_sh.at[pl.ds(row0, RPT_LAST)])


def _flush_slab(acc_sh, dst_fn, s):
    row0 = s * RPT

    @pl.when(s < NTILE - 1)
    def _():
        pltpu.sync_copy(acc_sh.at[pl.ds(row0, RPT)], dst_fn(row0, RPT))

    @pl.when(s == NTILE - 1)
    def _():
        pltpu.sync_copy(acc_sh.at[pl.ds(row0, RPT_LAST)],
                        dst_fn(row0, RPT_LAST))


def _degree_body(e_to, e_ti, e_ge, e_gl, e_bb, z1, ones1, padN, *rest):
    outs = rest[:10]
    ones_v, idx_v, acc_sh, zbuf_v, tbuf_v = rest[10:]
    c = lax.axis_index("c")
    s = lax.axis_index("s")
    row0 = s * RPT

    pltpu.sync_copy(ones1, ones_v)
    pltpu.sync_copy(z1, zbuf_v)

    edges = [e_to, e_ti, e_ge, e_gl, e_bb]
    hid = 0
    for r, (_, _, e_sz) in enumerate(RELSPEC):
        for side in range(2):
            nch = e_sz // CHD
            tail = e_sz % CHD
            base_off = side * e_sz

            @pl.when(hid % 2 == c)
            def _(hid=hid, r=r, nch=nch, tail=tail, base_off=base_off):
                e_ref = edges[r]

                @pl.when(s < NTILE - 1)
                def _():
                    pltpu.sync_copy(zbuf_v.at[pl.ds(0, RPT)],
                                    acc_sh.at[pl.ds(row0, RPT)])

                @pl.when(s == NTILE - 1)
                def _():
                    pltpu.sync_copy(zbuf_v.at[pl.ds(0, RPT_LAST)],
                                    acc_sh.at[pl.ds(row0, RPT_LAST)])

                plsc.subcore_barrier()

                def chunk(i, _):
                    k = s + i * NTILE
                    pltpu.sync_copy(e_ref.at[pl.ds(base_off + k * CHD, CHD)],
                                    idx_v)
                    pltpu.sync_copy(ones_v, acc_sh.at[idx_v], add=True)
                    return 0

                my_n = (nch - s + NTILE - 1) // NTILE
                lax.fori_loop(0, my_n, chunk, 0)
                if tail:
                    @pl.when(s == NTILE - 1)
                    def _():
                        pltpu.sync_copy(
                            e_ref.at[pl.ds(base_off + nch * CHD, tail)],
                            idx_v.at[pl.ds(0, tail)])
                        pltpu.sync_copy(padN.at[pl.ds(0, CHD - tail)],
                                        idx_v.at[pl.ds(tail, CHD - tail)])
                        pltpu.sync_copy(ones_v, acc_sh.at[idx_v], add=True)
                plsc.subcore_barrier()

                @pl.when(s < NTILE - 1)
                def _():
                    pltpu.sync_copy(acc_sh.at[pl.ds(row0, RPT)],
                                    tbuf_v.at[pl.ds(0, RPT)])
                    pltpu.sync_copy(tbuf_v.at[pl.ds(0, RPT)],
                                    outs[hid].at[pl.ds(row0, RPT)])

                @pl.when(s == NTILE - 1)
                def _():
                    pltpu.sync_copy(acc_sh.at[pl.ds(row0, RPT_LAST)],
                                    tbuf_v.at[pl.ds(0, RPT_LAST)])
                    pltpu.sync_copy(tbuf_v.at[pl.ds(0, RPT_LAST)],
                                    outs[hid].at[pl.ds(row0, RPT_LAST)])

                plsc.subcore_barrier()

            hid += 1


def _degree_kernel(e_to, e_ti, e_ge, e_gl, e_bb):
    z1 = jnp.zeros((RPT,), jnp.float32)
    ones1 = jnp.ones((CHD,), jnp.float32)
    padN = jnp.full((CHD,), N, jnp.int32)
    return pl.kernel(
        _degree_body,
        out_type=[jax.ShapeDtypeStruct((N,), jnp.float32)] * 10,
        mesh=_MESH,
        scratch_types=[
            pltpu.VMEM((CHD,), jnp.float32),
            pltpu.VMEM((CHD,), jnp.int32),
            pltpu.VMEM_SHARED((NPAD,), jnp.float32),
            pltpu.VMEM((RPT,), jnp.float32),
            pltpu.VMEM((RPT,), jnp.float32),
        ],
    )(e_to, e_ti, e_ge, e_gl, e_bb, z1, ones1, padN)


def _scatter_body(y_to, y_ti, y_ge, y_gl, y_bb, e_to, e_ti, e_ge, e_gl, e_bb,
                  zrows, pad0, padN, a_to, a_ti, a_ge, a_gl, a_bb, idx_s,
                  idx_d, rows_v, idx_s2, idx_d2, rows_v2, acc_sh, gsem,
                  gsem2, ssem, ssem2):
    c = lax.axis_index("c")
    s = lax.axis_index("s")
    w = s * NCORE + c

    ys = [y_to, y_ti, y_ge, y_gl, y_bb]
    es = [e_to, e_ti, e_ge, e_gl, e_bb]
    outs = [a_to, a_ti, a_ge, a_gl, a_bb]
    bufs = [(idx_s, idx_d, rows_v, gsem, ssem),
            (idx_s2, idx_d2, rows_v2, gsem2, ssem2)]
    for r, (_, _, e_sz) in enumerate(RELSPEC):
        nch = e_sz // CH
        tail = e_sz % CH
        y_ref, e_ref, o_ref = ys[r], es[r], outs[r]

        _zero_slab(zrows, acc_sh, s)
        plsc.subcore_barrier()

        my_n = (nch - w + NWORK - 1) // NWORK
        n_iter = my_n + (my_n % 2)

        def step(i, my_n, isx, idx, rvx, gsx, ssx, y_ref, e_ref, e_sz):
            @pl.when(i >= 2)
            def _():
                pltpu.make_async_copy(rvx, acc_sh.at[idx], ssx).wait()

            @pl.when(i < my_n)
            def _():
                k = w + i * NWORK
                pltpu.sync_copy(e_ref.at[pl.ds(k * CH, CH)], isx)
                pltpu.sync_copy(e_ref.at[pl.ds(e_sz + k * CH, CH)], idx)

            @pl.when(i >= my_n)
            def _():
                pltpu.sync_copy(pad0, isx)
                pltpu.sync_copy(padN, idx)

            pltpu.async_copy(y_ref.at[isx], rvx, gsx).wait()
            pltpu.async_copy(rvx, acc_sh.at[idx], ssx, add=True)

        def chunk(i, _, y_ref=y_ref, e_ref=e_ref, e_sz=e_sz, my_n=my_n):
            @pl.when(i % 2 == 0)
            def _():
                step(i, my_n, *bufs[0], y_ref, e_ref, e_sz)

            @pl.when(i % 2 == 1)
            def _():
                step(i, my_n, *bufs[1], y_ref, e_ref, e_sz)

            return 0

        lax.fori_loop(0, n_iter, chunk, 0)
        pltpu.make_async_copy(rows_v, acc_sh.at[idx_d], ssem).wait()
        pltpu.make_async_copy(rows_v2, acc_sh.at[idx_d2], ssem2).wait()
        if tail:
            @pl.when(w == NWORK - 1)
            def _(y_ref=y_ref, e_ref=e_ref, e_sz=e_sz, nch=nch, tail=tail):
                pltpu.sync_copy(e_ref.at[pl.ds(nch * CH, tail)],
                                idx_s.at[pl.ds(0, tail)])
                pltpu.sync_copy(e_ref.at[pl.ds(e_sz + nch * CH, tail)],
                                idx_d.at[pl.ds(0, tail)])
                pltpu.sync_copy(pad0.at[pl.ds(0, CH - tail)],
                                idx_s.at[pl.ds(tail, CH - tail)])
                pltpu.sync_copy(padN.at[pl.ds(0, CH - tail)],
                                idx_d.at[pl.ds(tail, CH - tail)])
                pltpu.async_copy(y_ref.at[idx_s], rows_v, gsem).wait()
                pltpu.sync_copy(rows_v, acc_sh.at[idx_d], add=True)
        plsc.subcore_barrier()
        _flush_slab(acc_sh, lambda r0, nr: o_ref.at[c, pl.ds(r0, nr)], s)
        plsc.subcore_barrier()


def _scatter_layer(y_to, y_ti, y_ge, y_gl, y_bb, e_to, e_ti, e_ge, e_gl,
                   e_bb):
    zrows = jnp.zeros((RPT, H), jnp.float32)
    pad0 = jnp.zeros((CH,), jnp.int32)
    padN = jnp.full((CH,), N, jnp.int32)
    acc = jax.ShapeDtypeStruct((NCORE, N, H), jnp.float32)
    return pl.kernel(
        _scatter_body,
        out_type=[acc] * 5,
        mesh=_MESH,
        scratch_types=[
            pltpu.VMEM((CH,), jnp.int32),
            pltpu.VMEM((CH,), jnp.int32),
            pltpu.VMEM((CH, H), jnp.float32),
            pltpu.VMEM((CH,), jnp.int32),
            pltpu.VMEM((CH,), jnp.int32),
            pltpu.VMEM((CH, H), jnp.float32),
            pltpu.VMEM_SHARED((NPAD, H), jnp.float32),
            pltpu.SemaphoreType.DMA,
            pltpu.SemaphoreType.DMA,
            pltpu.SemaphoreType.DMA,
            pltpu.SemaphoreType.DMA,
        ],
    )(y_to, y_ti, y_ge, y_gl, y_bb, e_to, e_ti, e_ge, e_gl, e_bb, zrows,
      pad0, padN)


_BM = 1000


def _ns(deg):
    return 1.0 / jnp.sqrt(jnp.maximum(deg, 1.0))


def _prologue_tc(nrel):
    def body(*refs):
        x, w1, b1, w2, b2 = refs[:5]
        pairs = refs[5:5 + 2 * nrel]
        youts = refs[5 + 2 * nrel:]
        h = jnp.dot(x[...], w1[...], precision=_HIGH) + b1[...]
        h = jax.nn.leaky_relu(h)
        h = jnp.dot(h, w2[...], precision=_HIGH) + b2[...]
        h = jax.nn.leaky_relu(h)
        for i in range(nrel):
            ds, wr = pairs[2 * i], pairs[2 * i + 1]
            youts[i][...] = jnp.dot(_ns(ds[...]) * h, wr[...],
                                    precision=_HIGH)

    def run(x, lin1, lin2, *pairs):
        in_specs = [
            pl.BlockSpec((_BM, H), lambda i: (i, 0)),
            pl.BlockSpec((H, HH), lambda i: (0, 0)),
            pl.BlockSpec((1, HH), lambda i: (0, 0)),
            pl.BlockSpec((HH, H), lambda i: (0, 0)),
            pl.BlockSpec((1, H), lambda i: (0, 0)),
        ]
        args = [x, lin1[0], lin1[1].reshape(1, HH), lin2[0],
                lin2[1].reshape(1, H)]
        for i in range(nrel):
            in_specs += [
                pl.BlockSpec((_BM, 1), lambda i: (i, 0)),
                pl.BlockSpec((H, H), lambda i: (0, 0)),
            ]
            args += [pairs[2 * i], pairs[2 * i + 1]]
        return pl.pallas_call(
            body,
            grid=(N // _BM,),
            in_specs=in_specs,
            out_specs=[pl.BlockSpec((_BM, H), lambda i: (i, 0))] * nrel,
            out_shape=[jax.ShapeDtypeStruct((N, H), jnp.float32)] * nrel,
        )(*args)

    return run


_prologue2 = _prologue_tc(2)
_prologue1 = _prologue_tc(1)


def _mid_tc(nacc, nrel, final=False):
    def body(*refs):
        accs = refs[:3 * nacc]
        rest = refs[3 * nacc:]
        acc_sum = None
        for i in range(nacc):
            a, d, b = accs[3 * i], accs[3 * i + 1], accs[3 * i + 2]
            av = a[...]
            t = _ns(d[...]) * (av[0] + av[1]) + b[...]
            acc_sum = t if acc_sum is None else acc_sum + t
        h = jax.nn.relu(acc_sum)
        if final:
            wout, bout, hg_out, pred_out = rest
            hg_out[...] = h
            pred_out[...] = jnp.dot(h, wout[...], precision=_HIGH) + bout[...]
        else:
            pairs = rest[:2 * nrel]
            youts = rest[2 * nrel:]
            for i in range(nrel):
                ds, wr = pairs[2 * i], pairs[2 * i + 1]
                youts[i][...] = jnp.dot(_ns(ds[...]) * h, wr[...],
                                        precision=_HIGH)

    def run(*args):
        in_specs = []
        flat = []
        for i in range(nacc):
            a, d, b = args[3 * i], args[3 * i + 1], args[3 * i + 2]
            in_specs += [
                pl.BlockSpec((NCORE, _BM, H), lambda i: (0, i, 0)),
                pl.BlockSpec((_BM, 1), lambda i: (i, 0)),
                pl.BlockSpec((1, H), lambda i: (0, 0)),
            ]
            flat += [a, d, b.reshape(1, H)]
        rest = args[3 * nacc:]
        if final:
            wout, bout = rest
            in_specs += [
                pl.BlockSpec((H, TGT), lambda i: (0, 0)),
                pl.BlockSpec((1, TGT), lambda i: (0, 0)),
            ]
            flat += [wout, bout.reshape(1, TGT)]
            out_specs = [
                pl.BlockSpec((_BM, H), lambda i: (i, 0)),
                pl.BlockSpec((_BM, TGT), lambda i: (i, 0)),
            ]
            out_shape = [
                jax.ShapeDtypeStruct((N, H), jnp.float32),
                jax.ShapeDtypeStruct((N, TGT), jnp.float32),
            ]
        else:
            for i in range(nrel):
                in_specs += [
                    pl.BlockSpec((_BM, 1), lambda i: (i, 0)),
                    pl.BlockSpec((H, H), lambda i: (0, 0)),
                ]
                flat += [rest[2 * i], rest[2 * i + 1]]
            out_specs = [pl.BlockSpec((_BM, H), lambda i: (i, 0))] * nrel
            out_shape = [jax.ShapeDtypeStruct((N, H), jnp.float32)] * nrel
        return pl.pallas_call(
            body,
            grid=(N // _BM,),
            in_specs=in_specs,
            out_specs=out_specs,
            out_shape=out_shape,
        )(*flat)

    return run


_mid_block = _mid_tc(1, 2)
_mid_net = _mid_tc(1, 2)
_mid_grid = _mid_tc(3, 1)
_final_grid = _mid_tc(3, 0, final=True)


def kernel(x_block, x_net, x_grid, ei_topo_out, ei_topo_in, ei_grid,
           ei_geom_loc, ei_geom_bb, params):
    p = params
    e_to = ei_topo_out.reshape(-1)
    e_ti = ei_topo_in.reshape(-1)
    e_ge = ei_grid.reshape(-1)
    e_gl = ei_geom_loc.reshape(-1)
    e_bb = ei_geom_bb.reshape(-1)

    degs = _degree_kernel(e_to, e_ti, e_ge, e_gl, e_bb)
    deg = jnp.stack(degs)
    d_src = {r[0]: deg[2 * i].reshape(N, 1) for i, r in enumerate(RELSPEC)}
    d_dst = {r[0]: deg[2 * i + 1].reshape(N, 1) for i, r in enumerate(RELSPEC)}

    def conv_w(l, r):
        return p['convs'][l][r][0]

    def conv_b(l, r):
        return p['convs'][l][r][1]

    y_to, y_gl = _prologue2(x_block, p['lin_block'], p['lin_block_2'],
                            d_src['topo_out'], conv_w(0, 'topo_out'),
                            d_src['geom_loc'], conv_w(0, 'geom_loc'))
    y_ti, y_bb = _prologue2(x_net, p['lin_net'], p['lin_net_2'],
                            d_src['topo_in'], conv_w(0, 'topo_in'),
                            d_src['geom_bb'], conv_w(0, 'geom_bb'))
    (y_ge,) = _prologue1(x_grid, p['lin_grid'], p['lin_grid_2'],
                         d_src['grid_edge'], conv_w(0, 'grid_edge'))

    for l in range(LAYERS):
        a_to, a_ti, a_ge, a_gl, a_bb = _scatter_layer(
            y_to, y_ti, y_ge, y_gl, y_bb, e_to, e_ti, e_ge, e_gl, e_bb)
        if l < LAYERS - 1:
            y_to, y_gl = _mid_block(
                a_ti, d_dst['topo_in'], conv_b(l, 'topo_in'),
                d_src['topo_out'], conv_w(l + 1, 'topo_out'),
                d_src['geom_loc'], conv_w(l + 1, 'geom_loc'))
            y_ti, y_bb = _mid_net(
                a_to, d_dst['topo_out'], conv_b(l, 'topo_out'),
                d_src['topo_in'], conv_w(l + 1, 'topo_in'),
                d_src['geom_bb'], conv_w(l + 1, 'geom_bb'))
            (y_ge,) = _mid_grid(
                a_ge, d_dst['grid_edge'], conv_b(l, 'grid_edge'),
                a_gl, d_dst['geom_loc'], conv_b(l, 'geom_loc'),
                a_bb, d_dst['geom_bb'], conv_b(l, 'geom_bb'),
                d_src['grid_edge'], conv_w(l + 1, 'grid_edge'))
        else:
            hg, pred = _final_grid(
                a_ge, d_dst['grid_edge'], conv_b(l, 'grid_edge'),
                a_gl, d_dst['geom_loc'], conv_b(l, 'geom_loc'),
                a_bb, d_dst['geom_bb'], conv_b(l, 'geom_bb'),
                p['out_lin'][0], p['out_lin'][1])
    return (pred, hg)


# speedup vs baseline: 1.6157x; 1.6157x over previous
import jax
import jax.numpy as jnp
from jax import lax
from jax.experimental import pallas as pl
from jax.experimental.pallas import tpu as pltpu
from jax.experimental.pallas import tpu_sc as plsc

N = 10000
H = 128
HH = H // 2
TGT = 4
LAYERS = 3

NTILE = 16
NCORE = 2
NWORK = NTILE * NCORE

RPT = 632
RPT_LAST = N - (NTILE - 1) * RPT

CH = 128
CHD = 128
NPAD = N + 8

RELSPEC = [
    ("topo_out", "block", 160000),
    ("topo_in", "net", 160000),
    ("grid_edge", "grid", 64000),
    ("geom_loc", "block", 40000),
    ("geom_bb", "net", 160000),
]

_MESH = plsc.VectorSubcoreMesh(core_axis_name="c", subcore_axis_name="s")
_HIGH = jax.lax.Precision.HIGHEST


def _zero_slab(zsrc, acc_sh, s):
    row0 = s * RPT

    @pl.when(s < NTILE - 1)
    def _():
        pltpu.sync_copy(zsrc.at[pl.ds(0, RPT)], acc_sh.at[pl.ds(row0, RPT)])

    @pl.when(s == NTILE - 1)
    def _():
        pltpu.sync_copy(zsrc.at[pl.ds(0, RPT_LAST)],
                        acc_sh.at[pl.ds(row0, RPT_LAST)])


def _flush_slab(acc_sh, dst_fn, s):
    row0 = s * RPT

    @pl.when(s < NTILE - 1)
    def _():
        pltpu.sync_copy(acc_sh.at[pl.ds(row0, RPT)], dst_fn(row0, RPT))

    @pl.when(s == NTILE - 1)
    def _():
        pltpu.sync_copy(acc_sh.at[pl.ds(row0, RPT_LAST)],
                        dst_fn(row0, RPT_LAST))


def _degree_body(e_to, e_ti, e_ge, e_gl, e_bb, z1, ones1, padN, *rest):
    outs = rest[:10]
    ones_v, idx_v, acc_sh, zbuf_v, tbuf_v = rest[10:]
    c = lax.axis_index("c")
    s = lax.axis_index("s")
    row0 = s * RPT

    pltpu.sync_copy(ones1, ones_v)
    pltpu.sync_copy(z1, zbuf_v)

    edges = [e_to, e_ti, e_ge, e_gl, e_bb]
    hid = 0
    for r, (_, _, e_sz) in enumerate(RELSPEC):
        for side in range(2):
            nch = e_sz // CHD
            tail = e_sz % CHD
            base_off = side * e_sz

            @pl.when(hid % 2 == c)
            def _(hid=hid, r=r, nch=nch, tail=tail, base_off=base_off):
                e_ref = edges[r]

                @pl.when(s < NTILE - 1)
                def _():
                    pltpu.sync_copy(zbuf_v.at[pl.ds(0, RPT)],
                                    acc_sh.at[pl.ds(row0, RPT)])

                @pl.when(s == NTILE - 1)
                def _():
                    pltpu.sync_copy(zbuf_v.at[pl.ds(0, RPT_LAST)],
                                    acc_sh.at[pl.ds(row0, RPT_LAST)])

                plsc.subcore_barrier()

                def chunk(i, _):
                    k = s + i * NTILE
                    pltpu.sync_copy(e_ref.at[pl.ds(base_off + k * CHD, CHD)],
                                    idx_v)
                    pltpu.sync_copy(ones_v, acc_sh.at[idx_v], add=True)
                    return 0

                my_n = (nch - s + NTILE - 1) // NTILE
                lax.fori_loop(0, my_n, chunk, 0)
                if tail:
                    @pl.when(s == NTILE - 1)
                    def _():
                        pltpu.sync_copy(
                            e_ref.at[pl.ds(base_off + nch * CHD, tail)],
                            idx_v.at[pl.ds(0, tail)])
                        pltpu.sync_copy(padN.at[pl.ds(0, CHD - tail)],
                                        idx_v.at[pl.ds(tail, CHD - tail)])
                        pltpu.sync_copy(ones_v, acc_sh.at[idx_v], add=True)
                plsc.subcore_barrier()

                @pl.when(s < NTILE - 1)
                def _():
                    pltpu.sync_copy(acc_sh.at[pl.ds(row0, RPT)],
                                    tbuf_v.at[pl.ds(0, RPT)])
                    pltpu.sync_copy(tbuf_v.at[pl.ds(0, RPT)],
                                    outs[hid].at[pl.ds(row0, RPT)])

                @pl.when(s == NTILE - 1)
                def _():
                    pltpu.sync_copy(acc_sh.at[pl.ds(row0, RPT_LAST)],
                                    tbuf_v.at[pl.ds(0, RPT_LAST)])
                    pltpu.sync_copy(tbuf_v.at[pl.ds(0, RPT_LAST)],
                                    outs[hid].at[pl.ds(row0, RPT_LAST)])

                plsc.subcore_barrier()

            hid += 1


def _degree_kernel(e_to, e_ti, e_ge, e_gl, e_bb):
    z1 = jnp.zeros((RPT,), jnp.float32)
    ones1 = jnp.ones((CHD,), jnp.float32)
    padN = jnp.full((CHD,), N, jnp.int32)
    return pl.kernel(
        _degree_body,
        out_type=[jax.ShapeDtypeStruct((N,), jnp.float32)] * 10,
        mesh=_MESH,
        scratch_types=[
            pltpu.VMEM((CHD,), jnp.float32),
            pltpu.VMEM((CHD,), jnp.int32),
            pltpu.VMEM_SHARED((NPAD,), jnp.float32),
            pltpu.VMEM((RPT,), jnp.float32),
            pltpu.VMEM((RPT,), jnp.float32),
        ],
    )(e_to, e_ti, e_ge, e_gl, e_bb, z1, ones1, padN)


def _scatter_body(y_to, y_ti, y_ge, y_gl, y_bb, e_to, e_ti, e_ge, e_gl, e_bb,
                  zrows, pad0, padN, a_to, a_ti, a_ge, a_gl, a_bb, idx_s,
                  idx_d, rows_v, acc_sh, gsem):
    c = lax.axis_index("c")
    s = lax.axis_index("s")
    w = s * NCORE + c

    ys = [y_to, y_ti, y_ge, y_gl, y_bb]
    es = [e_to, e_ti, e_ge, e_gl, e_bb]
    outs = [a_to, a_ti, a_ge, a_gl, a_bb]
    for r, (_, _, e_sz) in enumerate(RELSPEC):
        nch = e_sz // CH
        tail = e_sz % CH
        y_ref, e_ref, o_ref = ys[r], es[r], outs[r]

        _zero_slab(zrows, acc_sh, s)
        plsc.subcore_barrier()

        def chunk(i, _, y_ref=y_ref, e_ref=e_ref, e_sz=e_sz):
            k = w + i * NWORK
            pltpu.sync_copy(e_ref.at[pl.ds(k * CH, CH)], idx_s)
            pltpu.sync_copy(e_ref.at[pl.ds(e_sz + k * CH, CH)], idx_d)
            pltpu.async_copy(y_ref.at[idx_s], rows_v, gsem).wait()
            pltpu.sync_copy(rows_v, acc_sh.at[idx_d], add=True)
            return 0

        my_n = (nch - w + NWORK - 1) // NWORK
        lax.fori_loop(0, my_n, chunk, 0)
        if tail:
            @pl.when(w == NWORK - 1)
            def _(y_ref=y_ref, e_ref=e_ref, e_sz=e_sz, nch=nch, tail=tail):
                pltpu.sync_copy(e_ref.at[pl.ds(nch * CH, tail)],
                                idx_s.at[pl.ds(0, tail)])
                pltpu.sync_copy(e_ref.at[pl.ds(e_sz + nch * CH, tail)],
                                idx_d.at[pl.ds(0, tail)])
                pltpu.sync_copy(pad0.at[pl.ds(0, CH - tail)],
                                idx_s.at[pl.ds(tail, CH - tail)])
                pltpu.sync_copy(padN.at[pl.ds(0, CH - tail)],
                                idx_d.at[pl.ds(tail, CH - tail)])
                pltpu.async_copy(y_ref.at[idx_s], rows_v, gsem).wait()
                pltpu.sync_copy(rows_v, acc_sh.at[idx_d], add=True)
        plsc.subcore_barrier()
        _flush_slab(acc_sh, lambda r0, nr: o_ref.at[c, pl.ds(r0, nr)], s)
        plsc.subcore_barrier()


def _scatter_layer(y_to, y_ti, y_ge, y_gl, y_bb, e_to, e_ti, e_ge, e_gl,
                   e_bb):
    zrows = jnp.zeros((RPT, H), jnp.float32)
    pad0 = jnp.zeros((CH,), jnp.int32)
    padN = jnp.full((CH,), N, jnp.int32)
    acc = jax.ShapeDtypeStruct((NCORE, N, H), jnp.float32)
    return pl.kernel(
        _scatter_body,
        out_type=[acc] * 5,
        mesh=_MESH,
        scratch_types=[
            pltpu.VMEM((CH,), jnp.int32),
            pltpu.VMEM((CH,), jnp.int32),
            pltpu.VMEM((CH, H), jnp.float32),
            pltpu.VMEM_SHARED((NPAD, H), jnp.float32),
            pltpu.SemaphoreType.DMA,
        ],
    )(y_to, y_ti, y_ge, y_gl, y_bb, e_to, e_ti, e_ge, e_gl, e_bb, zrows,
      pad0, padN)


_BM = 1000


def _ns(deg):
    return 1.0 / jnp.sqrt(jnp.maximum(deg, 1.0))


def _prologue_tc(nrel):
    def body(*refs):
        x, w1, b1, w2, b2 = refs[:5]
        pairs = refs[5:5 + 2 * nrel]
        youts = refs[5 + 2 * nrel:]
        h = jnp.dot(x[...], w1[...], precision=_HIGH) + b1[...]
        h = jax.nn.leaky_relu(h)
        h = jnp.dot(h, w2[...], precision=_HIGH) + b2[...]
        h = jax.nn.leaky_relu(h)
        for i in range(nrel):
            ds, wr = pairs[2 * i], pairs[2 * i + 1]
            youts[i][...] = jnp.dot(_ns(ds[...]) * h, wr[...],
                                    precision=_HIGH)

    def run(x, lin1, lin2, *pairs):
        in_specs = [
            pl.BlockSpec((_BM, H), lambda i: (i, 0)),
            pl.BlockSpec((H, HH), lambda i: (0, 0)),
            pl.BlockSpec((1, HH), lambda i: (0, 0)),
            pl.BlockSpec((HH, H), lambda i: (0, 0)),
            pl.BlockSpec((1, H), lambda i: (0, 0)),
        ]
        args = [x, lin1[0], lin1[1].reshape(1, HH), lin2[0],
                lin2[1].reshape(1, H)]
        for i in range(nrel):
            in_specs += [
                pl.BlockSpec((_BM, 1), lambda i: (i, 0)),
                pl.BlockSpec((H, H), lambda i: (0, 0)),
            ]
            args += [pairs[2 * i], pairs[2 * i + 1]]
        return pl.pallas_call(
            body,
            grid=(N // _BM,),
            in_specs=in_specs,
            out_specs=[pl.BlockSpec((_BM, H), lambda i: (i, 0))] * nrel,
            out_shape=[jax.ShapeDtypeStruct((N, H), jnp.float32)] * nrel,
        )(*args)

    return run


_prologue2 = _prologue_tc(2)
_prologue1 = _prologue_tc(1)


def _mid_tc(nacc, nrel, final=False):
    def body(*refs):
        accs = refs[:3 * nacc]
        rest = refs[3 * nacc:]
        acc_sum = None
        for i in range(nacc):
            a, d, b = accs[3 * i], accs[3 * i + 1], accs[3 * i + 2]
            av = a[...]
            t = _ns(d[...]) * (av[0] + av[1]) + b[...]
            acc_sum = t if acc_sum is None else acc_sum + t
        h = jax.nn.relu(acc_sum)
        if final:
            wout, bout, hg_out, pred_out = rest
            hg_out[...] = h
            pred_out[...] = jnp.dot(h, wout[...], precision=_HIGH) + bout[...]
        else:
            pairs = rest[:2 * nrel]
            youts = rest[2 * nrel:]
            for i in range(nrel):
                ds, wr = pairs[2 * i], pairs[2 * i + 1]
                youts[i][...] = jnp.dot(_ns(ds[...]) * h, wr[...],
                                        precision=_HIGH)

    def run(*args):
        in_specs = []
        flat = []
        for i in range(nacc):
            a, d, b = args[3 * i], args[3 * i + 1], args[3 * i + 2]
            in_specs += [
                pl.BlockSpec((NCORE, _BM, H), lambda i: (0, i, 0)),
                pl.BlockSpec((_BM, 1), lambda i: (i, 0)),
                pl.BlockSpec((1, H), lambda i: (0, 0)),
            ]
            flat += [a, d, b.reshape(1, H)]
        rest = args[3 * nacc:]
        if final:
            wout, bout = rest
            in_specs += [
                pl.BlockSpec((H, TGT), lambda i: (0, 0)),
                pl.BlockSpec((1, TGT), lambda i: (0, 0)),
            ]
            flat += [wout, bout.reshape(1, TGT)]
            out_specs = [
                pl.BlockSpec((_BM, H), lambda i: (i, 0)),
                pl.BlockSpec((_BM, TGT), lambda i: (i, 0)),
            ]
            out_shape = [
                jax.ShapeDtypeStruct((N, H), jnp.float32),
                jax.ShapeDtypeStruct((N, TGT), jnp.float32),
            ]
        else:
            for i in range(nrel):
                in_specs += [
                    pl.BlockSpec((_BM, 1), lambda i: (i, 0)),
                    pl.BlockSpec((H, H), lambda i: (0, 0)),
                ]
                flat += [rest[2 * i], rest[2 * i + 1]]
            out_specs = [pl.BlockSpec((_BM, H), lambda i: (i, 0))] * nrel
            out_shape = [jax.ShapeDtypeStruct((N, H), jnp.float32)] * nrel
        return pl.pallas_call(
            body,
            grid=(N // _BM,),
            in_specs=in_specs,
            out_specs=out_specs,
            out_shape=out_shape,
        )(*flat)

    return run


_mid_block = _mid_tc(1, 2)
_mid_net = _mid_tc(1, 2)
_mid_grid = _mid_tc(3, 1)
_final_grid = _mid_tc(3, 0, final=True)


def kernel(x_block, x_net, x_grid, ei_topo_out, ei_topo_in, ei_grid,
           ei_geom_loc, ei_geom_bb, params):
    p = params
    e_to = ei_topo_out.reshape(-1)
    e_ti = ei_topo_in.reshape(-1)
    e_ge = ei_grid.reshape(-1)
    e_gl = ei_geom_loc.reshape(-1)
    e_bb = ei_geom_bb.reshape(-1)

    degs = _degree_kernel(e_to, e_ti, e_ge, e_gl, e_bb)
    deg = jnp.stack(degs)
    d_src = {r[0]: deg[2 * i].reshape(N, 1) for i, r in enumerate(RELSPEC)}
    d_dst = {r[0]: deg[2 * i + 1].reshape(N, 1) for i, r in enumerate(RELSPEC)}

    def conv_w(l, r):
        return p['convs'][l][r][0]

    def conv_b(l, r):
        return p['convs'][l][r][1]

    y_to, y_gl = _prologue2(x_block, p['lin_block'], p['lin_block_2'],
                            d_src['topo_out'], conv_w(0, 'topo_out'),
                            d_src['geom_loc'], conv_w(0, 'geom_loc'))
    y_ti, y_bb = _prologue2(x_net, p['lin_net'], p['lin_net_2'],
                            d_src['topo_in'], conv_w(0, 'topo_in'),
                            d_src['geom_bb'], conv_w(0, 'geom_bb'))
    (y_ge,) = _prologue1(x_grid, p['lin_grid'], p['lin_grid_2'],
                         d_src['grid_edge'], conv_w(0, 'grid_edge'))

    for l in range(LAYERS):
        a_to, a_ti, a_ge, a_gl, a_bb = _scatter_layer(
            y_to, y_ti, y_ge, y_gl, y_bb, e_to, e_ti, e_ge, e_gl, e_bb)
        if l < LAYERS - 1:
            y_to, y_gl = _mid_block(
                a_ti, d_dst['topo_in'], conv_b(l, 'topo_in'),
                d_src['topo_out'], conv_w(l + 1, 'topo_out'),
                d_src['geom_loc'], conv_w(l + 1, 'geom_loc'))
            y_ti, y_bb = _mid_net(
                a_to, d_dst['topo_out'], conv_b(l, 'topo_out'),
                d_src['topo_in'], conv_w(l + 1, 'topo_in'),
                d_src['geom_bb'], conv_w(l + 1, 'geom_bb'))
            (y_ge,) = _mid_grid(
                a_ge, d_dst['grid_edge'], conv_b(l, 'grid_edge'),
                a_gl, d_dst['geom_loc'], conv_b(l, 'geom_loc'),
                a_bb, d_dst['geom_bb'], conv_b(l, 'geom_bb'),
                d_src['grid_edge'], conv_w(l + 1, 'grid_edge'))
        else:
            hg, pred = _final_grid(
                a_ge, d_dst['grid_edge'], conv_b(l, 'grid_edge'),
                a_gl, d_dst['geom_loc'], conv_b(l, 'geom_loc'),
                a_bb, d_dst['geom_bb'], conv_b(l, 'geom_bb'),
                p['out_lin'][0], p['out_lin'][1])
    return (pred, hg)
